# baseline (speedup 1.0000x reference)
import jax
import jax.numpy as jnp
import numpy as np
from jax.experimental import pallas as pl
from jax.experimental.pallas import tpu as pltpu

_B, _LQ, _LD, _D, _NBINS = 32, 20, 4096, 300, 30
_NS = 4
_LS = _LD // _NS

_W32 = np.float32(2.0) / np.float32(_NBINS)
_EDGES = [float(np.float32(j) * _W32 - np.float32(1.0))
          for j in range(_NBINS + 1)]


def _quarter_counts(qn, ones_row, d, C):
    dsq = d * d
    nsq = jax.lax.dot_general(ones_row, dsq, (((1,), (1,)), ((), ())),
                              preferred_element_type=jnp.float32)
    dinv = jax.lax.rsqrt(nsq)
    s = jax.lax.dot_general(qn, d, (((1,), (1,)), ((), ())),
                            preferred_element_type=jnp.float32)
    x = s * dinv

    lane = jax.lax.broadcasted_iota(jnp.int32, (1, _NBINS + 1), 1)
    for j in range(_NBINS + 1):
        t = jnp.float32(_EDGES[j])
        m = (x > t) if j == _NBINS else (x >= t)
        cnt = jnp.sum(jnp.where(m, 1.0, 0.0), axis=1, keepdims=True)
        C = C + cnt * (lane == j).astype(jnp.float32)
    return C


def _drmm_body(ql_ref, sc_ref, q_ref, d0_ref, d1_ref, d2_ref, d3_ref,
               w1_ref, b1_ref, w2_ref, wg_ref, out_ref):
    b = pl.program_id(0)

    q = q_ref[0]
    qn = q * jax.lax.rsqrt(jnp.sum(q * q, axis=1, keepdims=True))
    ones_row = jnp.ones((1, _D), jnp.float32)

    C = jnp.zeros((_LQ, _NBINS + 1), jnp.float32)
    C = _quarter_counts(qn, ones_row, d0_ref[0], C)
    C = _quarter_counts(qn, ones_row, d1_ref[0], C)
    C = _quarter_counts(qn, ones_row, d2_ref[0], C)
    C = _quarter_counts(qn, ones_row, d3_ref[0], C)

    h = C[:, :_NBINS] - C[:, 1:_NBINS + 1]

    ql = ql_ref[b]
    row = jax.lax.broadcasted_iota(jnp.int32, (_LQ, 1), 0)
    h = h * (row < ql).astype(jnp.float32)
    h = jnp.log1p(h)

    z = jnp.tanh(jax.lax.dot_general(h, w1_ref[...], (((1,), (1,)), ((), ())),
                                     preferred_element_type=jnp.float32)
                 + b1_ref[...])
    z = jnp.tanh(jnp.sum(z * w2_ref[...], axis=1, keepdims=True) + sc_ref[0])
    z = jnp.tanh(z * sc_ref[2] + sc_ref[1])

    g = jnp.sum(qn * wg_ref[...], axis=1, keepdims=True) + sc_ref[3]
    g = g - jnp.max(g)
    e = jnp.exp(g)
    g = e / jnp.sum(e)

    out_ref[...] = jnp.sum(z * g, axis=(0, 1), keepdims=True).reshape(1, 1, 1)


def kernel(query, document, query_len, W1, b1, W2, b2, W3, b3, Wg, bg):
    scalars = jnp.concatenate([b2.reshape(1), b3.reshape(1),
                               W3.reshape(1), bg.reshape(1)])
    dspec = [pl.BlockSpec((1, _LS, _D), lambda b, ql, sc, i=i: (b, i, 0))
             for i in range(_NS)]
    grid_spec = pltpu.PrefetchScalarGridSpec(
        num_scalar_prefetch=2,
        grid=(_B,),
        in_specs=[
            pl.BlockSpec((1, _LQ, _D), lambda b, ql, sc: (b, 0, 0)),
            *dspec,
            pl.BlockSpec((5, _NBINS), lambda b, ql, sc: (0, 0)),
            pl.BlockSpec((1, 5), lambda b, ql, sc: (0, 0)),
            pl.BlockSpec((1, 5), lambda b, ql, sc: (0, 0)),
            pl.BlockSpec((1, _D), lambda b, ql, sc: (0, 0)),
        ],
        out_specs=pl.BlockSpec((1, 1, 1), lambda b, ql, sc: (b, 0, 0)),
    )
    out = pl.pallas_call(
        _drmm_body,
        grid_spec=grid_spec,
        out_shape=jax.ShapeDtypeStruct((_B, 1, 1), jnp.float32),
        compiler_params=pltpu.CompilerParams(
            dimension_semantics=("arbitrary",),
        ),
    )(query_len, scalars, query, document, document, document, document,
      W1, b1.reshape(1, 5), W2, Wg)
    return out[:, 0, 0]


# speedup vs baseline: 1.0553x; 1.0553x over previous
import jax
import jax.numpy as jnp
import numpy as np
from jax.experimental import pallas as pl
from jax.experimental.pallas import tpu as pltpu

_B, _LQ, _LD, _D, _NBINS = 32, 20, 4096, 300, 30

_W32 = np.float32(2.0) / np.float32(_NBINS)
_EDGES = [float(np.float32(j) * _W32 - np.float32(1.0))
          for j in range(_NBINS + 1)]
_SHIFT = 13


def _drmm_body(ql_ref, sc_ref, q_ref, d_ref, w1_ref, b1_ref,
               w2_ref, wg_ref, out_ref):
    b = pl.program_id(0)

    q = q_ref[0]
    qn = q * jax.lax.rsqrt(jnp.sum(q * q, axis=1, keepdims=True))
    d = d_ref[0]
    dinv = jax.lax.rsqrt(jnp.sum(d * d, axis=1))

    s = jax.lax.dot_general(qn, d, (((1,), (1,)), ((), ())),
                            preferred_element_type=jnp.float32)
    x = s * dinv[None, :]

    lane = jax.lax.broadcasted_iota(jnp.int32, (1, _NBINS + 1), 1)
    C = jnp.zeros((_LQ, _NBINS + 1), jnp.float32)
    i_one = jnp.int32(1)
    i_zero = jnp.int32(0)
    i_both = jnp.int32(1 + (1 << _SHIFT))
    for p in range(0, _NBINS + 1, 2):
        ta = jnp.float32(_EDGES[p])
        if p + 1 <= _NBINS:
            tb = jnp.float32(_EDGES[p + 1])
            mb = (x > tb) if p + 1 == _NBINS else (x >= tb)
            val = jnp.where(mb, i_both, jnp.where(x >= ta, i_one, i_zero))
            cnt = jnp.sum(val, axis=1, keepdims=True)
            cb = jnp.right_shift(cnt, _SHIFT)
            ca = jnp.bitwise_and(cnt, (1 << _SHIFT) - 1)
            C = C + (ca.astype(jnp.float32) * (lane == p).astype(jnp.float32)
                     + cb.astype(jnp.float32)
                     * (lane == p + 1).astype(jnp.float32))
        else:
            m = (x > ta)
            cnt = jnp.sum(jnp.where(m, i_one, i_zero), axis=1, keepdims=True)
            C = C + cnt.astype(jnp.float32) * (lane == p).astype(jnp.float32)

    h = C[:, :_NBINS] - C[:, 1:_NBINS + 1]

    ql = ql_ref[b]
    row = jax.lax.broadcasted_iota(jnp.int32, (_LQ, 1), 0)
    h = h * (row < ql).astype(jnp.float32)
    h = jnp.log1p(h)

    z = jnp.tanh(jax.lax.dot_general(h, w1_ref[...], (((1,), (1,)), ((), ())),
                                     preferred_element_type=jnp.float32)
                 + b1_ref[...])
    z = jnp.tanh(jnp.sum(z * w2_ref[...], axis=1, keepdims=True) + sc_ref[0])
    z = jnp.tanh(z * sc_ref[2] + sc_ref[1])

    g = jnp.sum(qn * wg_ref[...], axis=1, keepdims=True) + sc_ref[3]
    g = g - jnp.max(g)
    e = jnp.exp(g)
    g = e / jnp.sum(e)

    out_ref[...] = jnp.sum(z * g, axis=(0, 1), keepdims=True).reshape(1, 1, 1)


def kernel(query, document, query_len, W1, b1, W2, b2, W3, b3, Wg, bg):
    scalars = jnp.concatenate([b2.reshape(1), b3.reshape(1),
                               W3.reshape(1), bg.reshape(1)])
    grid_spec = pltpu.PrefetchScalarGridSpec(
        num_scalar_prefetch=2,
        grid=(_B,),
        in_specs=[
            pl.BlockSpec((1, _LQ, _D), lambda b, ql, sc: (b, 0, 0)),
            pl.BlockSpec((1, _LD, _D), lambda b, ql, sc: (b, 0, 0)),
            pl.BlockSpec((5, _NBINS), lambda b, ql, sc: (0, 0)),
            pl.BlockSpec((1, 5), lambda b, ql, sc: (0, 0)),
            pl.BlockSpec((1, 5), lambda b, ql, sc: (0, 0)),
            pl.BlockSpec((1, _D), lambda b, ql, sc: (0, 0)),
        ],
        out_specs=pl.BlockSpec((1, 1, 1), lambda b, ql, sc: (b, 0, 0)),
    )
    out = pl.pallas_call(
        _drmm_body,
        grid_spec=grid_spec,
        out_shape=jax.ShapeDtypeStruct((_B, 1, 1), jnp.float32),
        compiler_params=pltpu.CompilerParams(
            dimension_semantics=("arbitrary",),
        ),
    )(query_len, scalars, query, document,
      W1, b1.reshape(1, 5), W2, Wg)
    return out[:, 0, 0]


# speedup vs baseline: 1.0824x; 1.0257x over previous
import jax
import jax.numpy as jnp
import numpy as np
from jax.experimental import pallas as pl
from jax.experimental.pallas import tpu as pltpu

_B, _LQ, _LD, _D, _NBINS = 32, 20, 4096, 300, 30

_W32 = np.float32(2.0) / np.float32(_NBINS)
_EDGES = [float(np.float32(j) * _W32 - np.float32(1.0))
          for j in range(_NBINS + 1)]
_SHIFT = 13


def _score_one(ql, q, d, sc_ref, w1_ref, b1_ref, w2_ref, wg_ref):
    qn = q * jax.lax.rsqrt(jnp.sum(q * q, axis=1, keepdims=True))
    dinv = jax.lax.rsqrt(jnp.sum(d * d, axis=1))

    s = jax.lax.dot_general(qn, d, (((1,), (1,)), ((), ())),
                            preferred_element_type=jnp.float32)
    x = s * dinv[None, :]

    lane = jax.lax.broadcasted_iota(jnp.int32, (1, _NBINS + 1), 1)
    C = jnp.zeros((_LQ, _NBINS + 1), jnp.float32)
    i_one = jnp.int32(1)
    i_zero = jnp.int32(0)
    i_both = jnp.int32(1 + (1 << _SHIFT))
    for p in range(0, _NBINS + 1, 2):
        ta = jnp.float32(_EDGES[p])
        if p + 1 <= _NBINS:
            tb = jnp.float32(_EDGES[p + 1])
            mb = (x > tb) if p + 1 == _NBINS else (x >= tb)
            val = jnp.where(mb, i_both, jnp.where(x >= ta, i_one, i_zero))
            cnt = jnp.sum(val, axis=1, keepdims=True)
            cb = jnp.right_shift(cnt, _SHIFT)
            ca = jnp.bitwise_and(cnt, (1 << _SHIFT) - 1)
            C = C + (ca.astype(jnp.float32) * (lane == p).astype(jnp.float32)
                     + cb.astype(jnp.float32)
                     * (lane == p + 1).astype(jnp.float32))
        else:
            m = (x > ta)
            cnt = jnp.sum(jnp.where(m, i_one, i_zero), axis=1, keepdims=True)
            C = C + cnt.astype(jnp.float32) * (lane == p).astype(jnp.float32)

    h = C[:, :_NBINS] - C[:, 1:_NBINS + 1]

    row = jax.lax.broadcasted_iota(jnp.int32, (_LQ, 1), 0)
    h = h * (row < ql).astype(jnp.float32)
    h = jnp.log1p(h)

    z = jnp.tanh(jax.lax.dot_general(h, w1_ref[...], (((1,), (1,)), ((), ())),
                                     preferred_element_type=jnp.float32)
                 + b1_ref[...])
    z = jnp.tanh(jnp.sum(z * w2_ref[...], axis=1, keepdims=True) + sc_ref[0])
    z = jnp.tanh(z * sc_ref[2] + sc_ref[1])

    g = jnp.sum(qn * wg_ref[...], axis=1, keepdims=True) + sc_ref[3]
    g = g - jnp.max(g)
    e = jnp.exp(g)
    g = e / jnp.sum(e)

    return jnp.sum(z * g, axis=(0, 1), keepdims=True)


def _drmm_body(ql_ref, sc_ref, q_ref, d_ref, w1_ref, b1_ref,
               w2_ref, wg_ref, out_ref):
    b = pl.program_id(0)
    s0 = _score_one(ql_ref[2 * b], q_ref[0], d_ref[0],
                    sc_ref, w1_ref, b1_ref, w2_ref, wg_ref)
    s1 = _score_one(ql_ref[2 * b + 1], q_ref[1], d_ref[1],
                    sc_ref, w1_ref, b1_ref, w2_ref, wg_ref)
    out_ref[...] = jnp.concatenate([s0, s1], axis=0).reshape(2, 1, 1)


def kernel(query, document, query_len, W1, b1, W2, b2, W3, b3, Wg, bg):
    scalars = jnp.concatenate([b2.reshape(1), b3.reshape(1),
                               W3.reshape(1), bg.reshape(1)])
    grid_spec = pltpu.PrefetchScalarGridSpec(
        num_scalar_prefetch=2,
        grid=(_B // 2,),
        in_specs=[
            pl.BlockSpec((2, _LQ, _D), lambda b, ql, sc: (b, 0, 0)),
            pl.BlockSpec((2, _LD, _D), lambda b, ql, sc: (b, 0, 0)),
            pl.BlockSpec((5, _NBINS), lambda b, ql, sc: (0, 0)),
            pl.BlockSpec((1, 5), lambda b, ql, sc: (0, 0)),
            pl.BlockSpec((1, 5), lambda b, ql, sc: (0, 0)),
            pl.BlockSpec((1, _D), lambda b, ql, sc: (0, 0)),
        ],
        out_specs=pl.BlockSpec((2, 1, 1), lambda b, ql, sc: (b, 0, 0)),
    )
    out = pl.pallas_call(
        _drmm_body,
        grid_spec=grid_spec,
        out_shape=jax.ShapeDtypeStruct((_B, 1, 1), jnp.float32),
        compiler_params=pltpu.CompilerParams(
            dimension_semantics=("arbitrary",),
        ),
    )(query_len, scalars, query, document,
      W1, b1.reshape(1, 5), W2, Wg)
    return out[:, 0, 0]
